# baseline (speedup 1.0000x reference)
import jax
import jax.numpy as jnp
from jax.experimental import pallas as pl
from jax.experimental.pallas import tpu as pltpu

_TAPS = [(i, j) for i in range(3) for j in range(3)]
_NB = 8


def _net_kernel(xin_ref, wct_ref, mask_ref, bias_ref, out_ref, rhs_ref, p_ref):
    for t, (di, dj) in enumerate(_TAPS):
        s = 64 * di + dj
        rhs_ref[t] = xin_ref[:, s:s + 4352] * mask_ref[0][None, :]
    for c in range(_NB // 2):
        p_ref[c % 2] = jax.lax.dot_general(
            wct_ref[:], rhs_ref[:, 2 * c:2 * c + 2, :],
            dimension_numbers=(((1,), (0,)), ((), ())),
            preferred_element_type=jnp.float32)
        for h in range(2):
            b = 2 * c + h
            slabs = [p_ref[c % 2, 32 * t:32 * t + 32, h,
                           64 * ei + ej:64 * ei + ej + 4096]
                     for t, (ei, ej) in enumerate(_TAPS)]
            z = ((bias_ref[:] + slabs[0]) + (slabs[1] + slabs[2])) \
                + ((slabs[3] + slabs[4]) + (slabs[5] + slabs[6])) \
                + (slabs[7] + slabs[8])
            out_ref[b] = z


def kernel(x, W1, b1, W2, b2):
    n = x.shape[0]
    xin = jnp.pad(x.reshape(n, 4096), ((0, 0), (130, 382))).astype(jnp.bfloat16)

    w2r = W2.reshape(9, 64, 32)
    wct = jnp.einsum("uc,tco->tou", W1.reshape(9, 64), w2r).reshape(288, 9).astype(jnp.bfloat16)

    ll = jnp.arange(4352)
    mask = ((ll >= 130) & (ll < 4098) & ((ll - 130) % 64 < 62))
    mask = mask.astype(jnp.bfloat16)[None, :]

    bt = jnp.einsum('c,tco->to', b1, w2r)
    pq = jnp.arange(64)
    plane = jnp.zeros((32, 64, 64), jnp.float32) + b2[:, None, None]
    for t, (ei, ej) in enumerate(_TAPS):
        rowok = (pq + ei - 2 >= 0) & (pq + ei - 2 <= 61)
        colok = (pq + ej - 2 >= 0) & (pq + ej - 2 <= 61)
        m = (rowok[:, None] & colok[None, :]).astype(jnp.float32)
        plane = plane + bt[t][:, None, None] * m[None, :, :]
    bias = plane.reshape(32, 4096)

    out = pl.pallas_call(
        _net_kernel,
        grid=(n // _NB,),
        in_specs=[
            pl.BlockSpec((_NB, 4608), lambda i: (i, 0)),
            pl.BlockSpec((288, 9), lambda i: (0, 0)),
            pl.BlockSpec((1, 4352), lambda i: (0, 0)),
            pl.BlockSpec((32, 4096), lambda i: (0, 0)),
        ],
        out_specs=pl.BlockSpec((_NB, 32, 4096), lambda i: (i, 0, 0)),
        out_shape=jax.ShapeDtypeStruct((n, 32, 4096), jnp.float32),
        scratch_shapes=[pltpu.VMEM((9, _NB, 4352), jnp.bfloat16),
                        pltpu.VMEM((2, 288, 2, 4352), jnp.float32)],
    )(xin, wct, mask, bias)
    return out.reshape(n, 32, 64, 64)


# speedup vs baseline: 2.7684x; 2.7684x over previous
import jax
import jax.numpy as jnp
from jax.experimental import pallas as pl
from jax.experimental.pallas import tpu as pltpu

_TAPS = [(i, j) for i in range(3) for j in range(3)]
_NB = 8


def _net_kernel(xin_ref, wct_ref, mask_ref, bias_ref, out_ref, rhs_ref, xk_ref):
    @pl.when(pl.program_id(0) == 0)
    def _zero_junk_rows():
        xk_ref[...] = jnp.zeros((2, 144, 4096), jnp.bfloat16)

    for b in range(_NB):
        for t, (di, dj) in enumerate(_TAPS):
            s = 64 * di + dj
            rhs_ref[t, b * 4352:b * 4352 + 4352] = (
                xin_ref[b, s:s + 4352] * mask_ref[0])
    for b in range(_NB):
        rb = b % 2
        for t, (ei, ej) in enumerate(_TAPS):
            s = b * 4352 + 64 * ei + ej
            xk_ref[rb, 16 * t:16 * t + 9, :] = rhs_ref[:, s:s + 4096]
        z = bias_ref[:] + jnp.dot(wct_ref[:], xk_ref[rb],
                                  preferred_element_type=jnp.float32)
        out_ref[b] = z


def kernel(x, W1, b1, W2, b2):
    n = x.shape[0]
    xin = jnp.pad(x.reshape(n, 4096), ((0, 0), (130, 382))).astype(jnp.bfloat16)

    w2r = W2.reshape(9, 64, 32)
    wc = jnp.einsum('uc,tco->otu', W1.reshape(9, 64), w2r)
    wct = jnp.zeros((32, 9, 16), jnp.float32)
    wct = wct.at[:, :, :9].set(wc).reshape(32, 144).astype(jnp.bfloat16)

    ll = jnp.arange(4352)
    mask = ((ll >= 130) & (ll < 4098) & ((ll - 130) % 64 < 62))
    mask = mask.astype(jnp.bfloat16)[None, :]

    bt = jnp.einsum('c,tco->to', b1, w2r)
    pq = jnp.arange(64)
    plane = jnp.zeros((32, 64, 64), jnp.float32) + b2[:, None, None]
    for t, (ei, ej) in enumerate(_TAPS):
        rowok = (pq + ei - 2 >= 0) & (pq + ei - 2 <= 61)
        colok = (pq + ej - 2 >= 0) & (pq + ej - 2 <= 61)
        m = (rowok[:, None] & colok[None, :]).astype(jnp.float32)
        plane = plane + bt[t][:, None, None] * m[None, :, :]
    bias = plane.reshape(32, 4096)

    out = pl.pallas_call(
        _net_kernel,
        grid=(n // _NB,),
        in_specs=[
            pl.BlockSpec((_NB, 4608), lambda i: (i, 0)),
            pl.BlockSpec((32, 144), lambda i: (0, 0)),
            pl.BlockSpec((1, 4352), lambda i: (0, 0)),
            pl.BlockSpec((32, 4096), lambda i: (0, 0)),
        ],
        out_specs=pl.BlockSpec((_NB, 32, 4096), lambda i: (i, 0, 0)),
        out_shape=jax.ShapeDtypeStruct((n, 32, 4096), jnp.float32),
        scratch_shapes=[pltpu.VMEM((9, _NB * 4352), jnp.bfloat16),
                        pltpu.VMEM((2, 144, 4096), jnp.bfloat16)],
    )(xin, wct, mask, bias)
    return out.reshape(n, 32, 64, 64)


# speedup vs baseline: 2.7900x; 1.0078x over previous
import jax
import jax.numpy as jnp
from jax.experimental import pallas as pl
from jax.experimental.pallas import tpu as pltpu

_TAPS = [(i, j) for i in range(3) for j in range(3)]
_NB = 16


def _net_kernel(xin_ref, wct_ref, mask_ref, bias_ref, out_ref, rhs_ref, xk_ref):
    @pl.when(pl.program_id(0) == 0)
    def _zero_junk_rows():
        xk_ref[...] = jnp.zeros((2, 144, 4096), jnp.bfloat16)

    for b in range(_NB):
        for t, (di, dj) in enumerate(_TAPS):
            s = 64 * di + dj
            rhs_ref[t, b * 4352:b * 4352 + 4352] = (
                xin_ref[b, s:s + 4352] * mask_ref[0])
    for b in range(_NB):
        rb = b % 2
        for t, (ei, ej) in enumerate(_TAPS):
            s = b * 4352 + 64 * ei + ej
            xk_ref[rb, 16 * t:16 * t + 9, :] = rhs_ref[:, s:s + 4096]
        z = bias_ref[:] + jnp.dot(wct_ref[:], xk_ref[rb],
                                  preferred_element_type=jnp.float32)
        out_ref[b] = z


def kernel(x, W1, b1, W2, b2):
    n = x.shape[0]
    xin = jnp.pad(x.reshape(n, 4096), ((0, 0), (130, 382))).astype(jnp.bfloat16)

    w2r = W2.reshape(9, 64, 32)
    wc = jnp.einsum('uc,tco->otu', W1.reshape(9, 64), w2r)
    wct = jnp.zeros((32, 9, 16), jnp.float32)
    wct = wct.at[:, :, :9].set(wc).reshape(32, 144).astype(jnp.bfloat16)

    ll = jnp.arange(4352)
    mask = ((ll >= 130) & (ll < 4098) & ((ll - 130) % 64 < 62))
    mask = mask.astype(jnp.bfloat16)[None, :]

    bt = jnp.einsum('c,tco->to', b1, w2r)
    pq = jnp.arange(64)
    plane = jnp.zeros((32, 64, 64), jnp.float32) + b2[:, None, None]
    for t, (ei, ej) in enumerate(_TAPS):
        rowok = (pq + ei - 2 >= 0) & (pq + ei - 2 <= 61)
        colok = (pq + ej - 2 >= 0) & (pq + ej - 2 <= 61)
        m = (rowok[:, None] & colok[None, :]).astype(jnp.float32)
        plane = plane + bt[t][:, None, None] * m[None, :, :]
    bias = plane.reshape(32, 4096)

    out = pl.pallas_call(
        _net_kernel,
        grid=(n // _NB,),
        in_specs=[
            pl.BlockSpec((_NB, 4608), lambda i: (i, 0)),
            pl.BlockSpec((32, 144), lambda i: (0, 0)),
            pl.BlockSpec((1, 4352), lambda i: (0, 0)),
            pl.BlockSpec((32, 4096), lambda i: (0, 0)),
        ],
        out_specs=pl.BlockSpec((_NB, 32, 4096), lambda i: (i, 0, 0)),
        out_shape=jax.ShapeDtypeStruct((n, 32, 4096), jnp.float32),
        scratch_shapes=[pltpu.VMEM((9, _NB * 4352), jnp.bfloat16),
                        pltpu.VMEM((2, 144, 4096), jnp.bfloat16)],
    )(xin, wct, mask, bias)
    return out.reshape(n, 32, 64, 64)


# speedup vs baseline: 2.7946x; 1.0016x over previous
import jax
import jax.numpy as jnp
from jax.experimental import pallas as pl
from jax.experimental.pallas import tpu as pltpu

_TAPS = [(i, j) for i in range(3) for j in range(3)]
_NB = 32


def _net_kernel(xin_ref, wct_ref, mask_ref, bias_ref, out_ref, rhs_ref, xk_ref):
    @pl.when(pl.program_id(0) == 0)
    def _zero_junk_rows():
        xk_ref[...] = jnp.zeros((2, 144, 4096), jnp.bfloat16)

    for b in range(_NB):
        for t, (di, dj) in enumerate(_TAPS):
            s = 64 * di + dj
            rhs_ref[t, b * 4352:b * 4352 + 4352] = (
                xin_ref[b, s:s + 4352] * mask_ref[0])
    for b in range(_NB):
        rb = b % 2
        for t, (ei, ej) in enumerate(_TAPS):
            s = b * 4352 + 64 * ei + ej
            xk_ref[rb, 16 * t:16 * t + 9, :] = rhs_ref[:, s:s + 4096]
        z = bias_ref[:] + jnp.dot(wct_ref[:], xk_ref[rb],
                                  preferred_element_type=jnp.float32)
        out_ref[b] = z


def kernel(x, W1, b1, W2, b2):
    n = x.shape[0]
    xin = jnp.pad(x.reshape(n, 4096), ((0, 0), (130, 382))).astype(jnp.bfloat16)

    w2r = W2.reshape(9, 64, 32)
    wc = jnp.einsum('uc,tco->otu', W1.reshape(9, 64), w2r)
    wct = jnp.zeros((32, 9, 16), jnp.float32)
    wct = wct.at[:, :, :9].set(wc).reshape(32, 144).astype(jnp.bfloat16)

    ll = jnp.arange(4352)
    mask = ((ll >= 130) & (ll < 4098) & ((ll - 130) % 64 < 62))
    mask = mask.astype(jnp.bfloat16)[None, :]

    bt = jnp.einsum('c,tco->to', b1, w2r)
    pq = jnp.arange(64)
    plane = jnp.zeros((32, 64, 64), jnp.float32) + b2[:, None, None]
    for t, (ei, ej) in enumerate(_TAPS):
        rowok = (pq + ei - 2 >= 0) & (pq + ei - 2 <= 61)
        colok = (pq + ej - 2 >= 0) & (pq + ej - 2 <= 61)
        m = (rowok[:, None] & colok[None, :]).astype(jnp.float32)
        plane = plane + bt[t][:, None, None] * m[None, :, :]
    bias = plane.reshape(32, 4096)

    out = pl.pallas_call(
        _net_kernel,
        grid=(n // _NB,),
        in_specs=[
            pl.BlockSpec((_NB, 4608), lambda i: (i, 0)),
            pl.BlockSpec((32, 144), lambda i: (0, 0)),
            pl.BlockSpec((1, 4352), lambda i: (0, 0)),
            pl.BlockSpec((32, 4096), lambda i: (0, 0)),
        ],
        out_specs=pl.BlockSpec((_NB, 32, 4096), lambda i: (i, 0, 0)),
        out_shape=jax.ShapeDtypeStruct((n, 32, 4096), jnp.float32),
        scratch_shapes=[pltpu.VMEM((9, _NB * 4352), jnp.bfloat16),
                        pltpu.VMEM((2, 144, 4096), jnp.bfloat16)],
    )(xin, wct, mask, bias)
    return out.reshape(n, 32, 64, 64)


# speedup vs baseline: 2.7999x; 1.0019x over previous
import jax
import jax.numpy as jnp
from jax.experimental import pallas as pl
from jax.experimental.pallas import tpu as pltpu

_TAPS = [(i, j) for i in range(3) for j in range(3)]
_NB = 32


def _net_kernel(xin_ref, wct_ref, mask_ref, bias_ref, out_ref, rhs_ref, xk_ref):
    @pl.when(pl.program_id(0) == 0)
    def _zero_junk_rows():
        xk_ref[...] = jnp.zeros((4, 144, 4096), jnp.bfloat16)

    for b in range(_NB):
        for t, (di, dj) in enumerate(_TAPS):
            s = 64 * di + dj
            rhs_ref[t, b * 4352:b * 4352 + 4352] = (
                xin_ref[b, s:s + 4352] * mask_ref[0])
    for b in range(_NB):
        rb = b % 4
        for t, (ei, ej) in enumerate(_TAPS):
            s = b * 4352 + 64 * ei + ej
            xk_ref[rb, 16 * t:16 * t + 9, :] = rhs_ref[:, s:s + 4096]
        z = bias_ref[:] + jnp.dot(wct_ref[:], xk_ref[rb],
                                  preferred_element_type=jnp.float32)
        out_ref[b] = z


def kernel(x, W1, b1, W2, b2):
    n = x.shape[0]
    xin = jnp.pad(x.reshape(n, 4096), ((0, 0), (130, 382))).astype(jnp.bfloat16)

    w2r = W2.reshape(9, 64, 32)
    wc = jnp.einsum('uc,tco->otu', W1.reshape(9, 64), w2r)
    wct = jnp.zeros((32, 9, 16), jnp.float32)
    wct = wct.at[:, :, :9].set(wc).reshape(32, 144).astype(jnp.bfloat16)

    ll = jnp.arange(4352)
    mask = ((ll >= 130) & (ll < 4098) & ((ll - 130) % 64 < 62))
    mask = mask.astype(jnp.bfloat16)[None, :]

    bt = jnp.einsum('c,tco->to', b1, w2r)
    pq = jnp.arange(64)
    plane = jnp.zeros((32, 64, 64), jnp.float32) + b2[:, None, None]
    for t, (ei, ej) in enumerate(_TAPS):
        rowok = (pq + ei - 2 >= 0) & (pq + ei - 2 <= 61)
        colok = (pq + ej - 2 >= 0) & (pq + ej - 2 <= 61)
        m = (rowok[:, None] & colok[None, :]).astype(jnp.float32)
        plane = plane + bt[t][:, None, None] * m[None, :, :]
    bias = plane.reshape(32, 4096)

    out = pl.pallas_call(
        _net_kernel,
        grid=(n // _NB,),
        in_specs=[
            pl.BlockSpec((_NB, 4608), lambda i: (i, 0)),
            pl.BlockSpec((32, 144), lambda i: (0, 0)),
            pl.BlockSpec((1, 4352), lambda i: (0, 0)),
            pl.BlockSpec((32, 4096), lambda i: (0, 0)),
        ],
        out_specs=pl.BlockSpec((_NB, 32, 4096), lambda i: (i, 0, 0)),
        out_shape=jax.ShapeDtypeStruct((n, 32, 4096), jnp.float32),
        scratch_shapes=[pltpu.VMEM((9, _NB * 4352), jnp.bfloat16),
                        pltpu.VMEM((4, 144, 4096), jnp.bfloat16)],
    )(xin, wct, mask, bias)
    return out.reshape(n, 32, 64, 64)
